# baseline (speedup 1.0000x reference)
import functools

import jax
import jax.numpy as jnp
from jax import lax
from jax.experimental import pallas as pl
from jax.experimental.pallas import tpu as pltpu
from jax.experimental.pallas import tpu_sc as plsc

_NUM_CORES = 2
_NUM_SUBCORES = 16
_NUM_WORKERS = _NUM_CORES * _NUM_SUBCORES
_LANES = 16


@functools.cache
def _make_reverse_kernel(M, N, R, depth, nbo):
    rows_per_w = M // _NUM_WORKERS
    nblk = rows_per_w // R
    assert nblk % depth == 0 and depth % nbo == 0

    mesh = plsc.VectorSubcoreMesh(core_axis_name="c", subcore_axis_name="s")

    scratch = ([pltpu.VMEM((R, N), jnp.float32)] * (depth + nbo)
               + [pltpu.SemaphoreType.DMA] * (depth + nbo))

    @functools.partial(
        pl.kernel,
        out_type=jax.ShapeDtypeStruct((M, N), jnp.float32),
        mesh=mesh,
        scratch_types=scratch,
    )
    def k(in_hbm, out_hbm, *refs):
        bi = refs[:depth]
        bo = refs[depth:depth + nbo]
        si = refs[depth + nbo:2 * depth + nbo]
        so = refs[2 * depth + nbo:2 * depth + 2 * nbo]

        wid = lax.axis_index("s") * _NUM_CORES + lax.axis_index("c")
        base_row = wid * rows_per_w

        def start_in(blk, d):
            pltpu.async_copy(
                in_hbm.at[pl.ds(base_row + blk * R, R)], bi[d], si[d])

        def wait_in(d):
            pltpu.make_async_copy(
                in_hbm.at[pl.ds(base_row, R)], bi[d], si[d]).wait()

        def start_out(blk, s):
            pltpu.async_copy(
                bo[s], out_hbm.at[pl.ds(base_row + blk * R, R)], so[s])

        def wait_out(s):
            pltpu.make_async_copy(
                bo[s], out_hbm.at[pl.ds(base_row, R)], so[s]).wait()

        def compute(buf_in, buf_out):
            for r in range(R):
                @plsc.parallel_loop(0, N, step=_LANES, unroll=8)
                def _(c):
                    v = buf_in[r, pl.ds(N - _LANES - c, _LANES)]
                    buf_out[r, pl.ds(c, _LANES)] = lax.rev(v, dimensions=(0,))

        for d in range(depth):
            start_in(d, d)

        def group_body(g, carry):
            blk0 = depth * g
            for d in range(depth):
                b = blk0 + d
                s = d % nbo
                wait_in(d)

                if d >= nbo:
                    wait_out(s)
                else:
                    @pl.when(g > 0)
                    def _():
                        wait_out(s)

                compute(bi[d], bo[s])
                start_out(b, s)

                @pl.when(b + depth < nblk)
                def _():
                    start_in(b + depth, d)
            return carry

        lax.fori_loop(0, nblk // depth, group_body, 0)
        for s in range(nbo):
            wait_out(s)

    return k


def kernel(inputs):
    M, N = inputs.shape
    return _make_reverse_kernel(M, N, 4, 4, 2)(inputs)


# speedup vs baseline: 1.0157x; 1.0157x over previous
import functools

import jax
import jax.numpy as jnp
from jax import lax
from jax.experimental import pallas as pl
from jax.experimental.pallas import tpu as pltpu
from jax.experimental.pallas import tpu_sc as plsc

_NUM_CORES = 2
_NUM_SUBCORES = 16
_NUM_WORKERS = _NUM_CORES * _NUM_SUBCORES
_LANES = 16


@functools.cache
def _make_reverse_kernel(M, N, R, depth, nbo):
    rows_per_w = M // _NUM_WORKERS
    nblk = rows_per_w // R
    assert nblk % depth == 0 and depth % nbo == 0

    mesh = plsc.VectorSubcoreMesh(core_axis_name="c", subcore_axis_name="s")

    scratch = ([pltpu.VMEM((R, N), jnp.float32)] * (depth + nbo)
               + [pltpu.SemaphoreType.DMA] * (depth + nbo))

    @functools.partial(
        pl.kernel,
        out_type=jax.ShapeDtypeStruct((M, N), jnp.float32),
        mesh=mesh,
        scratch_types=scratch,
    )
    def k(in_hbm, out_hbm, *refs):
        bi = refs[:depth]
        bo = refs[depth:depth + nbo]
        si = refs[depth + nbo:2 * depth + nbo]
        so = refs[2 * depth + nbo:2 * depth + 2 * nbo]

        wid = lax.axis_index("s") * _NUM_CORES + lax.axis_index("c")
        base_row = wid * rows_per_w

        def start_in(blk, d):
            pltpu.async_copy(
                in_hbm.at[pl.ds(base_row + blk * R, R)], bi[d], si[d])

        def wait_in(d):
            pltpu.make_async_copy(
                in_hbm.at[pl.ds(base_row, R)], bi[d], si[d]).wait()

        def start_out(blk, s):
            pltpu.async_copy(
                bo[s], out_hbm.at[pl.ds(base_row + blk * R, R)], so[s])

        def wait_out(s):
            pltpu.make_async_copy(
                bo[s], out_hbm.at[pl.ds(base_row, R)], so[s]).wait()

        def compute(buf_in, buf_out):
            for r in range(R):
                @plsc.parallel_loop(0, N, step=_LANES, unroll=8)
                def _(c):
                    v = buf_in[r, pl.ds(N - _LANES - c, _LANES)]
                    buf_out[r, pl.ds(c, _LANES)] = lax.rev(v, dimensions=(0,))

        for d in range(depth):
            start_in(d, d)

        def group_body(g, carry):
            blk0 = depth * g
            for d in range(depth):
                b = blk0 + d
                s = d % nbo
                wait_in(d)

                if d >= nbo:
                    wait_out(s)
                else:
                    @pl.when(g > 0)
                    def _():
                        wait_out(s)

                compute(bi[d], bo[s])
                start_out(b, s)

                @pl.when(b + depth < nblk)
                def _():
                    start_in(b + depth, d)
            return carry

        lax.fori_loop(0, nblk // depth, group_body, 0)
        for s in range(nbo):
            wait_out(s)

    return k


def kernel(inputs):
    M, N = inputs.shape
    return _make_reverse_kernel(M, N, 2, 8, 4)(inputs)
